# baseline (speedup 1.0000x reference)
import jax
import jax.numpy as jnp
from jax.experimental import pallas as pl
from jax.experimental.pallas import tpu as pltpu

_SEQ_BLOCK = 2048


def _add_kernel(emb_ref, pos_ref, out_ref):
    out_ref[...] = emb_ref[...] + pos_ref[...]


def kernel(embeddings, pos_table):
    batch, seq, hid = embeddings.shape
    seq_block = 512
    grid = (seq // seq_block,)
    return pl.pallas_call(
        _add_kernel,
        grid=grid,
        in_specs=[
            pl.BlockSpec((batch, seq_block, hid), lambda i: (0, i, 0)),
            pl.BlockSpec((seq_block, hid), lambda i: (i, 0)),
        ],
        out_specs=pl.BlockSpec((batch, seq_block, hid), lambda i: (0, i, 0)),
        out_shape=jax.ShapeDtypeStruct((batch, seq, hid), embeddings.dtype),
        compiler_params=pltpu.CompilerParams(
            dimension_semantics=("arbitrary",),
        ),
    )(embeddings, pos_table)


# speedup vs baseline: 1.0048x; 1.0048x over previous
import jax
import jax.numpy as jnp
from jax.experimental import pallas as pl
from jax.experimental.pallas import tpu as pltpu

_SEQ_BLOCK = 2048


def _add_kernel(emb_ref, pos_ref, out_ref):
    out_ref[...] = emb_ref[...] + pos_ref[...]


def kernel(embeddings, pos_table):
    batch, seq, hid = embeddings.shape
    grid = (seq // _SEQ_BLOCK, batch)
    return pl.pallas_call(
        _add_kernel,
        grid=grid,
        in_specs=[
            pl.BlockSpec((1, _SEQ_BLOCK, hid), lambda i, j: (j, i, 0)),
            pl.BlockSpec((_SEQ_BLOCK, hid), lambda i, j: (i, 0)),
        ],
        out_specs=pl.BlockSpec((1, _SEQ_BLOCK, hid), lambda i, j: (j, i, 0)),
        out_shape=jax.ShapeDtypeStruct((batch, seq, hid), embeddings.dtype),
        compiler_params=pltpu.CompilerParams(
            dimension_semantics=("arbitrary", "arbitrary"),
        ),
    )(embeddings, pos_table)
